# baseline (speedup 1.0000x reference)
import functools

import jax
import jax.numpy as jnp
from jax import lax
from jax.experimental import pallas as pl
from jax.experimental.pallas import tpu as pltpu
from jax.experimental.pallas import tpu_sc as plsc

N_ROWS = 1_000_000
N_CLS = 64

BLK_ROWS = 4000
N_BLKS = N_ROWS // BLK_ROWS

SC_NC = 2
SC_NS = 16
SC_L = 16
NW = SC_NC * SC_NS
CHUNK = 31248
COPY_LEN = 31312
N_GROUPS = COPY_LEN // SC_L


def _argmax_body(x_ref, out_ref):
    x = x_ref[0]
    m = jnp.max(x, axis=1, keepdims=True)
    col = lax.broadcasted_iota(jnp.int32, x.shape, 1)
    out_ref[0, 0, :] = jnp.min(jnp.where(x == m, col, N_CLS), axis=1)


_argmax_call = pl.pallas_call(
    _argmax_body,
    grid=(N_BLKS,),
    in_specs=[pl.BlockSpec((1, BLK_ROWS, N_CLS), lambda i: (i, 0, 0))],
    out_specs=pl.BlockSpec((1, 1, BLK_ROWS), lambda i: (i, 0, 0)),
    out_shape=jax.ShapeDtypeStruct((N_BLKS, 1, BLK_ROWS), jnp.int32),
)


def _sc_hist_body(idx_hbm, out_hbm, idx_v, acc_v, part_v):
    wid = lax.axis_index("s") * SC_NC + lax.axis_index("c")
    base = pl.multiple_of(wid * CHUNK, 8)
    pltpu.sync_copy(idx_hbm.at[pl.ds(base, COPY_LEN)], idx_v)

    zeros16 = jnp.zeros((SC_L,), jnp.float32)
    for i in range(SC_L * N_CLS // SC_L):
        acc_v[pl.ds(i * SC_L, SC_L)] = zeros16

    lane = lax.iota(jnp.int32, SC_L)
    lane_off = lane * N_CLS
    ones16 = jnp.ones((SC_L,), jnp.float32)
    limit = jnp.where(wid == NW - 1, COPY_LEN, CHUNK)

    def body(g, carry):
        offs = g * SC_L
        iv = idx_v[pl.ds(offs, SC_L)]
        mask = (lane + offs) < limit
        plsc.addupdate_scatter(acc_v, [iv + lane_off], ones16, mask=mask)
        return carry

    lax.fori_loop(0, N_GROUPS, body, 0)

    for cg in range(N_CLS // SC_L):
        s = acc_v[pl.ds(cg * SC_L, SC_L)]
        for r in range(1, SC_L):
            s = s + acc_v[pl.ds(r * N_CLS + cg * SC_L, SC_L)]
        part_v[pl.ds(cg * SC_L, SC_L)] = s

    pltpu.sync_copy(part_v, out_hbm.at[wid])


def _sc_hist_call(idx):
    call = functools.partial(
        pl.kernel,
        mesh=plsc.VectorSubcoreMesh(
            core_axis_name="c", subcore_axis_name="s",
            num_cores=SC_NC, num_subcores=SC_NS,
        ),
        out_type=jax.ShapeDtypeStruct((NW, N_CLS), jnp.float32),
        scratch_types=[
            pltpu.VMEM((COPY_LEN,), jnp.int32),
            pltpu.VMEM((SC_L * N_CLS,), jnp.float32),
            pltpu.VMEM((N_CLS,), jnp.float32),
        ],
        compiler_params=pltpu.CompilerParams(needs_layout_passes=False),
    )(_sc_hist_body)
    return call(idx)


def _finish_body(p_ref, out_ref):
    out_ref[...] = jnp.sum(p_ref[...], axis=0, keepdims=True) * (1.0 / N_ROWS)


_finish_call = pl.pallas_call(
    _finish_body,
    out_shape=jax.ShapeDtypeStruct((1, N_CLS), jnp.float32),
)


@jax.jit
def kernel(input):
    x3 = input.reshape(N_BLKS, BLK_ROWS, N_CLS)
    idx = _argmax_call(x3).reshape(N_ROWS)
    parts = _sc_hist_call(idx)
    return _finish_call(parts).reshape(N_CLS)


# speedup vs baseline: 5.4781x; 5.4781x over previous
import functools

import jax
import jax.numpy as jnp
from jax import lax
from jax.experimental import pallas as pl
from jax.experimental.pallas import tpu as pltpu
from jax.experimental.pallas import tpu_sc as plsc

N_ROWS = 1_000_000
N_CLS = 64

BLK_ROWS = 8192
N_BLKS = -(-N_ROWS // BLK_ROWS)

SC_NC = 2
SC_NS = 16
SC_L = 16
NW = SC_NC * SC_NS
CHUNK = 31248
COPY_LEN = 31312
N_GROUPS = COPY_LEN // SC_L


def _argmax_body(x_ref, out_ref):
    x = x_ref[...]
    m = jnp.max(x, axis=0, keepdims=True)
    cls = lax.broadcasted_iota(jnp.int32, x.shape, 0)
    out_ref[...] = jnp.min(jnp.where(x == m, cls, N_CLS), axis=0)


_argmax_call = pl.pallas_call(
    _argmax_body,
    grid=(N_BLKS,),
    in_specs=[pl.BlockSpec((N_CLS, BLK_ROWS), lambda i: (0, i))],
    out_specs=pl.BlockSpec((BLK_ROWS,), lambda i: (i,)),
    out_shape=jax.ShapeDtypeStruct((N_ROWS,), jnp.int32),
)


def _sc_hist_body(idx_hbm, out_hbm, idx_v, acc_v, part_v):
    wid = lax.axis_index("s") * SC_NC + lax.axis_index("c")
    base = pl.multiple_of(wid * CHUNK, 8)
    pltpu.sync_copy(idx_hbm.at[pl.ds(base, COPY_LEN)], idx_v)

    zeros16 = jnp.zeros((SC_L,), jnp.float32)
    for i in range(SC_L * N_CLS // SC_L):
        acc_v[pl.ds(i * SC_L, SC_L)] = zeros16

    lane = lax.iota(jnp.int32, SC_L)
    lane_off = lane * N_CLS
    ones16 = jnp.ones((SC_L,), jnp.float32)
    limit = jnp.where(wid == NW - 1, COPY_LEN, CHUNK)

    def body(g, carry):
        offs = g * SC_L
        iv = idx_v[pl.ds(offs, SC_L)]
        mask = (lane + offs) < limit
        plsc.addupdate_scatter(acc_v, [iv + lane_off], ones16, mask=mask)
        return carry

    lax.fori_loop(0, N_GROUPS, body, 0)

    for cg in range(N_CLS // SC_L):
        s = acc_v[pl.ds(cg * SC_L, SC_L)]
        for r in range(1, SC_L):
            s = s + acc_v[pl.ds(r * N_CLS + cg * SC_L, SC_L)]
        part_v[pl.ds(cg * SC_L, SC_L)] = s

    pltpu.sync_copy(part_v, out_hbm.at[wid])


def _sc_hist_call(idx):
    call = functools.partial(
        pl.kernel,
        mesh=plsc.VectorSubcoreMesh(
            core_axis_name="c", subcore_axis_name="s",
            num_cores=SC_NC, num_subcores=SC_NS,
        ),
        out_type=jax.ShapeDtypeStruct((NW, N_CLS), jnp.float32),
        scratch_types=[
            pltpu.VMEM((COPY_LEN,), jnp.int32),
            pltpu.VMEM((SC_L * N_CLS,), jnp.float32),
            pltpu.VMEM((N_CLS,), jnp.float32),
        ],
        compiler_params=pltpu.CompilerParams(needs_layout_passes=False),
    )(_sc_hist_body)
    return call(idx)


def _finish_body(p_ref, out_ref):
    out_ref[...] = jnp.sum(p_ref[...], axis=0, keepdims=True) * (1.0 / N_ROWS)


_finish_call = pl.pallas_call(
    _finish_body,
    out_shape=jax.ShapeDtypeStruct((1, N_CLS), jnp.float32),
)


@jax.jit
def kernel(input):
    idx = _argmax_call(input.T)
    parts = _sc_hist_call(idx)
    return _finish_call(parts).reshape(N_CLS)
